# Initial kernel scaffold; baseline (speedup 1.0000x reference)
import jax
import jax.numpy as jnp
from jax.experimental import pallas as pl


def kernel(x, edge_index, LP_W, LP_b, W1_0, W1_1, b1, W2_0, W2_1, b2, W3_0, W3_1, b3):
    raise NotImplementedError("write your pallas kernel here")


# speedup vs baseline: 12.4498x; 12.4498x over previous
import jax
import jax.numpy as jnp
from jax import lax
from jax.experimental import pallas as pl
from jax.experimental.pallas import tpu as pltpu
from jax.experimental.pallas import tpu_sc as plsc

N = 248
NP = 256
F = 512
E = 15872
CW = N * NP
_VEC = 16


def _sc_body(edge_hbm, c_hbm, src_v, dst_v, c_v):
    cid = lax.axis_index("c")
    sid = lax.axis_index("s")

    @pl.when(jnp.logical_and(cid == 0, sid == 0))
    def _():
        pltpu.sync_copy(edge_hbm.at[0], src_v)
        pltpu.sync_copy(edge_hbm.at[1], dst_v)

        def zero_body(j, carry):
            c_v[pl.ds(j * _VEC, _VEC)] = jnp.zeros((_VEC,), jnp.float32)
            return carry

        lax.fori_loop(0, CW // _VEC, zero_body, 0)

        def edge_body(j, carry):
            s = src_v[pl.ds(j * _VEC, _VEC)]
            d = dst_v[pl.ds(j * _VEC, _VEC)]
            idx = d * NP + s
            val = jnp.where(s != d, 1.0, 0.0).astype(jnp.float32)
            plsc.addupdate_scatter(c_v, [idx], val)
            return carry

        lax.fori_loop(0, E // _VEC, edge_body, 0)

        pltpu.sync_copy(c_v, c_hbm)


import functools


@functools.cache
def _sc_build_c():
    return pl.kernel(
        _sc_body,
        out_type=jax.ShapeDtypeStruct((CW,), jnp.float32),
        mesh=plsc.VectorSubcoreMesh(core_axis_name="c", subcore_axis_name="s"),
        compiler_params=pltpu.CompilerParams(needs_layout_passes=False),
        scratch_types=[
            pltpu.VMEM((E,), jnp.int32),
            pltpu.VMEM((E,), jnp.int32),
            pltpu.VMEM((CW,), jnp.float32),
        ],
    )


def _tc_body(c_ref, x_ref, lpw_ref, lpb_ref, w10_ref, w11_ref, b1_ref,
             w20_ref, w21_ref, b2_ref, w30_ref, w31_ref, b3_ref, out_ref):
    f32 = jnp.float32
    C = c_ref[...]
    deg = jnp.sum(C, axis=0, keepdims=True)
    dis = jnp.where(deg > 0, lax.rsqrt(jnp.maximum(deg, 1e-12)), 0.0)

    rows = lax.broadcasted_iota(jnp.int32, (NP, NP), 0)
    cols = lax.broadcasted_iota(jnp.int32, (NP, NP), 1)
    eye = jnp.where(rows == cols, 1.0, 0.0).astype(f32)
    ddiag = eye * dis
    A = -jnp.dot(ddiag, C * dis, preferred_element_type=f32)

    rowmask = jnp.where(rows < N, 1.0, 0.0).astype(f32)

    def mm_t(a, w):
        return lax.dot_general(a, w, (((1,), (1,)), ((), ())),
                               preferred_element_type=f32)

    def silu(h):
        return h / (1.0 + jnp.exp(-h))

    def ln(h):
        mu = jnp.sum(h * rowmask) / (N * NP)
        dcen = (h - mu) * rowmask
        var = jnp.sum(dcen * dcen) / (N * NP)
        return (h - mu) * lax.rsqrt(var + 1e-5)

    x = x_ref[...]
    res = mm_t(x, lpw_ref[...]) + lpb_ref[...]
    t1 = jnp.dot(A, x, preferred_element_type=f32)
    h = mm_t(x, w10_ref[...]) + mm_t(t1, w11_ref[...]) + b1_ref[...]
    h = ln(silu(h))
    t2 = jnp.dot(A, h, preferred_element_type=f32)
    h = res + mm_t(h, w20_ref[...]) + mm_t(t2, w21_ref[...]) + b2_ref[...]
    h = ln(silu(h))
    t3 = jnp.dot(A, h, preferred_element_type=f32)
    out_ref[...] = (mm_t(h, w30_ref[...]) + mm_t(t3, w31_ref[...])
                    + b3_ref[...])


_tc_call = pl.pallas_call(
    _tc_body,
    out_shape=jax.ShapeDtypeStruct((NP, 128), jnp.float32),
)


def kernel(x, edge_index, LP_W, LP_b, W1_0, W1_1, b1, W2_0, W2_1, b2,
           W3_0, W3_1, b3):
    c_flat = _sc_build_c()(edge_index)
    cpad = jnp.pad(c_flat.reshape(N, NP), ((0, NP - N), (0, 0)))
    xpad = jnp.pad(x, ((0, NP - N), (0, 0)))
    out = _tc_call(cpad, xpad, LP_W, LP_b.reshape(1, -1), W1_0, W1_1,
                   b1.reshape(1, -1), W2_0, W2_1, b2.reshape(1, -1),
                   W3_0, W3_1, b3.reshape(1, -1))
    return out[:N]


# speedup vs baseline: 24.8408x; 1.9953x over previous
import functools

import jax
import jax.numpy as jnp
from jax import lax
from jax.experimental import pallas as pl
from jax.experimental.pallas import tpu as pltpu
from jax.experimental.pallas import tpu_sc as plsc

N = 248
NP = 256
F = 512
E = 15872
CW = N * NP
_VEC = 16

_NSUB = 16
_EPC = E // 2
_EPT = _EPC // _NSUB
_NV = _EPT // _VEC
_IDXROWS = 4
_ZW = CW // _NSUB


def _sc_body(edge_hbm, c_hbm, src_v, dst_v, idx_v, val_v, zero_v, c_sh):
    cid = lax.axis_index("c")
    sid = lax.axis_index("s")
    base = cid * _EPC + sid * _EPT

    pltpu.sync_copy(edge_hbm.at[pl.ds(base, _EPT)], src_v)
    pltpu.sync_copy(edge_hbm.at[pl.ds(E + base, _EPT)], dst_v)

    def zero_body(i, carry):
        for k in range(8):
            zero_v[pl.ds((i * 8 + k) * _VEC, _VEC)] = jnp.zeros(
                (_VEC,), jnp.float32)
        return carry

    lax.fori_loop(0, _ZW // (8 * _VEC), zero_body, 0)
    pltpu.sync_copy(zero_v, c_sh.at[pl.ds(sid * _ZW, _ZW)])

    for j in range(_NV):
        s = src_v[pl.ds(j * _VEC, _VEC)]
        d = dst_v[pl.ds(j * _VEC, _VEC)]
        row, col = divmod(j * _VEC, 128)
        idx_v[row, pl.ds(col, _VEC)] = d * NP + s
        val_v[row, pl.ds(col, _VEC)] = jnp.where(
            s != d, 1.0, 0.0).astype(jnp.float32)
    pad = _NV * _VEC - (_IDXROWS - 1) * 128
    idx_v[_IDXROWS - 1, pl.ds(pad, 128 - pad)] = jnp.zeros(
        (128 - pad,), jnp.int32)
    val_v[_IDXROWS - 1, pl.ds(pad, 128 - pad)] = jnp.zeros(
        (128 - pad,), jnp.float32)

    plsc.subcore_barrier()

    for j in range(_IDXROWS):
        pltpu.sync_copy(val_v.at[j], c_sh.at[idx_v.at[j]], add=True)

    plsc.subcore_barrier()

    @pl.when(sid == 0)
    def _():
        pltpu.sync_copy(c_sh, c_hbm.at[pl.ds(cid * CW, CW)])


@functools.cache
def _sc_build_c():
    return pl.kernel(
        _sc_body,
        out_type=jax.ShapeDtypeStruct((2 * CW,), jnp.float32),
        mesh=plsc.VectorSubcoreMesh(core_axis_name="c", subcore_axis_name="s"),
        compiler_params=pltpu.CompilerParams(needs_layout_passes=False),
        scratch_types=[
            pltpu.VMEM((_EPT,), jnp.int32),
            pltpu.VMEM((_EPT,), jnp.int32),
            pltpu.VMEM((_IDXROWS, 128), jnp.int32),
            pltpu.VMEM((_IDXROWS, 128), jnp.float32),
            pltpu.VMEM((_ZW,), jnp.float32),
            pltpu.VMEM_SHARED((CW,), jnp.float32),
        ],
    )


def _tc_body(c_ref, x_ref, lpw_ref, lpb_ref, w10_ref, w11_ref, b1_ref,
             w20_ref, w21_ref, b2_ref, w30_ref, w31_ref, b3_ref, out_ref):
    f32 = jnp.float32
    C = c_ref[:N, :] + c_ref[N:, :]
    deg = jnp.sum(C, axis=0, keepdims=True)
    dis = jnp.where(deg > 0, lax.rsqrt(jnp.maximum(deg, 1e-12)), 0.0)

    rows = lax.broadcasted_iota(jnp.int32, (N, N), 0)
    cols = lax.broadcasted_iota(jnp.int32, (N, N), 1)
    eye = jnp.where(rows == cols, 1.0, 0.0).astype(f32)
    ddiag = eye * dis[:, :N]
    A = -jnp.dot(ddiag, (C * dis)[:, :N], preferred_element_type=f32)

    def mm_t(a, w):
        return lax.dot_general(a, w, (((1,), (1,)), ((), ())),
                               preferred_element_type=f32)

    def silu(h):
        return h / (1.0 + jnp.exp(-h))

    def ln(h):
        mu = jnp.sum(h) / (N * 256)
        dcen = h - mu
        var = jnp.sum(dcen * dcen) / (N * 256)
        return dcen * lax.rsqrt(var + 1e-5)

    x = x_ref[...]
    res = mm_t(x, lpw_ref[...]) + lpb_ref[...]
    t1 = jnp.dot(A, x, preferred_element_type=f32)
    h = mm_t(x, w10_ref[...]) + mm_t(t1, w11_ref[...]) + b1_ref[...]
    h = ln(silu(h))
    t2 = jnp.dot(A, h, preferred_element_type=f32)
    h = res + mm_t(h, w20_ref[...]) + mm_t(t2, w21_ref[...]) + b2_ref[...]
    h = ln(silu(h))
    t3 = jnp.dot(A, h, preferred_element_type=f32)
    out_ref[...] = (mm_t(h, w30_ref[...]) + mm_t(t3, w31_ref[...])
                    + b3_ref[...])


_tc_call = pl.pallas_call(
    _tc_body,
    out_shape=jax.ShapeDtypeStruct((N, 128), jnp.float32),
)


def kernel(x, edge_index, LP_W, LP_b, W1_0, W1_1, b1, W2_0, W2_1, b2,
           W3_0, W3_1, b3):
    c2 = _sc_build_c()(edge_index.reshape(-1)).reshape(2 * N, NP)
    return _tc_call(c2, x, LP_W, LP_b.reshape(1, -1), W1_0, W1_1,
                    b1.reshape(1, -1), W2_0, W2_1, b2.reshape(1, -1),
                    W3_0, W3_1, b3.reshape(1, -1))
